# baseline (speedup 1.0000x reference)
import functools

import jax
import jax.numpy as jnp
from jax import lax
from jax.experimental import pallas as pl
from jax.experimental.pallas import tpu as pltpu
from jax.experimental.pallas import tpu_sc as plsc

NSTEPS = 10
HSTEP = 1.0 / NSTEPS
NTILES = 16
H = 5

_MESH = plsc.VectorSubcoreMesh(core_axis_name="c", subcore_axis_name="s",
                               num_cores=1)
_SC_PARAMS = pltpu.CompilerParams(needs_layout_passes=False)


def _round_up(v, m):
    return (v + m - 1) // m * m


def _make_bucket_kernel(e, n_pad, rpt, cap, chunk):
    nvec = chunk // 16

    def body(src_hbm, dst_hbm, deg_hbm, srco_hbm, dsto_hbm,
             srcc_v, dstc_v, srcb_v, dstb_v, deg_v):
        t = lax.axis_index("s")
        lo = t * rpt
        lo_v = jnp.full((16,), lo, jnp.int32)
        hi_v = jnp.full((16,), lo + rpt, jnp.int32)
        ones = jnp.full((16,), 1.0, jnp.float32)
        zeros = jnp.zeros((16,), jnp.float32)

        def zb(i, _):
            deg_v[pl.ds(i * 16, 16)] = zeros
            return ()
        lax.fori_loop(0, rpt // 16, zb, ())

        padsrc = jnp.full((16,), (n_pad - 1) + (H - 1) * rpt * (NTILES - 1),
                          jnp.int32)
        zoff = jnp.zeros((16,), jnp.int32)

        def pb(i, _):
            srcb_v[pl.ds(i * 16, 16)] = padsrc
            dstb_v[pl.ds(i * 16, 16)] = zoff
            return ()
        lax.fori_loop(0, cap // 16, pb, ())

        nchunks = e // chunk

        def cb(c, off):
            pltpu.sync_copy(src_hbm.at[pl.ds(c * chunk, chunk)], srcc_v)
            pltpu.sync_copy(dst_hbm.at[pl.ds(c * chunk, chunk)], dstc_v)

            def vb(i, off):
                s = srcc_v[pl.ds(i * 16, 16)]
                d = dstc_v[pl.ds(i * 16, 16)]
                m = (d >= lo_v) & (d < hi_v)
                m = m & (jnp.full((16,), off, jnp.int32) <
                         jnp.full((16,), cap - 16, jnp.int32))
                dloc = d - lo_v
                g = s + jnp.full((16,), (H - 1) * rpt, jnp.int32) * (
                    s // jnp.full((16,), rpt, jnp.int32))
                plsc.addupdate_scatter(deg_v, [dloc], ones, mask=m)
                plsc.store_compressed(srcb_v.at[pl.ds(off, 16)], g, mask=m)
                plsc.store_compressed(dstb_v.at[pl.ds(off, 16)], dloc, mask=m)
                return off + jnp.sum(m.astype(jnp.int32))
            return lax.fori_loop(0, nvec, vb, off)
        lax.fori_loop(0, nchunks, cb, jnp.int32(0))

        pltpu.sync_copy(deg_v, deg_hbm.at[pl.ds(lo, rpt)])
        pltpu.sync_copy(srcb_v, srco_hbm.at[pl.ds(t * cap, cap)])
        pltpu.sync_copy(dstb_v, dsto_hbm.at[pl.ds(t * cap, cap)])

    return pl.kernel(
        body,
        out_type=[jax.ShapeDtypeStruct((n_pad,), jnp.float32),
                  jax.ShapeDtypeStruct((NTILES * cap,), jnp.int32),
                  jax.ShapeDtypeStruct((NTILES * cap,), jnp.int32)],
        mesh=_MESH,
        compiler_params=_SC_PARAMS,
        scratch_types=[pltpu.VMEM((chunk,), jnp.int32),
                       pltpu.VMEM((chunk,), jnp.int32),
                       pltpu.VMEM((cap,), jnp.int32),
                       pltpu.VMEM((cap,), jnp.int32),
                       pltpu.VMEM((rpt,), jnp.float32)],
    )


def _tc_front(n, x_ref, w1_ref, deg_ref, z_ref, zt_ref, dinv_ref):
    n_pad = deg_ref.shape[1]
    z = lax.dot_general(w1_ref[...], x_ref[...],
                        (((0,), (1,)), ((), ())),
                        preferred_element_type=jnp.float32)
    deg = deg_ref[...] + 1.0
    col = lax.broadcasted_iota(jnp.int32, (1, n_pad), 1)
    dinv = jnp.where(col < n, lax.rsqrt(jnp.maximum(deg, 1e-12)), 0.0)
    z_ref[...] = z
    zt_ref[...] = z * dinv
    dinv_ref[...] = dinv


def _make_ode_kernel(n_pad, rpt, cap):
    nev = cap // 16
    nch = rpt // 16

    blk = H * rpt

    def body(z_hbm, zt_hbm, dinv_hbm, srco_hbm, dsto_hbm, consts_hbm,
             sc_hbm, *scr):
        (table,
         raw0, raw1, raw2, raw3, raw4,
         zs0, zs1, zs2, zs3, zs4,
         sa0, sa1, sa2, sa3, sa4,
         ac0, ac1, ac2, ac3, ac4,
         srcb, dstb, dinv_v, cons_v,
         shA, shB) = scr
        raw = (raw0, raw1, raw2, raw3, raw4)
        zs = (zs0, zs1, zs2, zs3, zs4)
        sacc = (sa0, sa1, sa2, sa3, sa4)
        stac = (ac0, ac1, ac2, ac3, ac4)

        t = lax.axis_index("s")
        ob = t * rpt
        tb = t * blk
        zeros = jnp.zeros((16,), jnp.float32)

        pltpu.sync_copy(srco_hbm.at[pl.ds(t * cap, cap)], srcb)
        pltpu.sync_copy(dsto_hbm.at[pl.ds(t * cap, cap)], dstb)
        pltpu.sync_copy(dinv_hbm.at[pl.ds(ob, rpt)], dinv_v)
        pltpu.sync_copy(consts_hbm, cons_v)
        pltpu.sync_copy(zt_hbm, table)
        for j in range(H):
            pltpu.sync_copy(z_hbm.at[pl.ds(j * n_pad + ob, rpt)], zs[j])

        def zinit(i, _):
            o = i * 16
            for j in range(H):
                raw[j][pl.ds(o, 16)] = zeros
                sacc[j][pl.ds(o, 16)] = zeros
                stac[j][pl.ds(o, 16)] = zeros
            return ()
        lax.fori_loop(0, nch, zinit, ())

        def msplat(i, j):
            return cons_v[pl.ds((i * H + j) * 16, 16)]

        def bzsplat(j):
            return cons_v[pl.ds((25 + j) * 16, 16)]

        def b1splat(j):
            return cons_v[pl.ds((30 + j) * 16, 16)]

        def agg():
            @plsc.parallel_loop(0, nev, unroll=8)
            def _(i):
                o = i * 16
                g = srcb[pl.ds(o, 16)]
                d = dstb[pl.ds(o, 16)]
                for j in range(H):
                    gj = g if j == 0 else g + jnp.full((16,), j * rpt,
                                                       jnp.int32)
                    v = plsc.load_gather(table, [gj])
                    plsc.addupdate_scatter(raw[j], [d], v)

        def exchange(shared):
            pltpu.sync_copy(table.at[pl.ds(tb, blk)],
                            shared.at[pl.ds(tb, blk)])
            plsc.subcore_barrier()
            pltpu.sync_copy(shared, table)

        def post1(i):
            o = i * 16
            dv = dinv_v[pl.ds(o, 16)]
            for j in range(H):
                r = raw[j][pl.ds(o, 16)]
                utj = table[pl.ds(tb + j * rpt + o, 16)]
                out1 = dv * (r + utj) + b1splat(j)
                hre = jnp.maximum(out1, 0.0)
                table[pl.ds(tb + j * rpt + o, 16)] = dv * hre
                raw[j][pl.ds(o, 16)] = zeros

        def make_post2(w, cnext):
            wv = jnp.float32(w)
            cv = jnp.float32(cnext)

            def post2(i):
                o = i * 16
                dv = dinv_v[pl.ds(o, 16)]
                out2 = []
                for j in range(H):
                    r = raw[j][pl.ds(o, 16)]
                    htj = table[pl.ds(tb + j * rpt + o, 16)]
                    o2 = dv * (r + htj)
                    out2.append(o2)
                    raw[j][pl.ds(o, 16)] = zeros
                    stac[j][pl.ds(o, 16)] = stac[j][pl.ds(o, 16)] + wv * o2
                for j in range(H):
                    kj = bzsplat(j)
                    for i2 in range(H):
                        kj = kj + out2[i2] * msplat(i2, j)
                    zin = zs[j][pl.ds(o, 16)] + cv * kj
                    table[pl.ds(tb + j * rpt + o, 16)] = dv * zin
            return post2

        def post4(i):
            h6 = jnp.float32(HSTEP / 6.0)
            hv = jnp.float32(HSTEP)
            o = i * 16
            dv = dinv_v[pl.ds(o, 16)]
            sa = []
            for j in range(H):
                r = raw[j][pl.ds(o, 16)]
                htj = table[pl.ds(tb + j * rpt + o, 16)]
                o2 = dv * (r + htj)
                raw[j][pl.ds(o, 16)] = zeros
                sav = stac[j][pl.ds(o, 16)] + o2
                sa.append(sav)
                stac[j][pl.ds(o, 16)] = zeros
                sacc[j][pl.ds(o, 16)] = sacc[j][pl.ds(o, 16)] + h6 * sav
            for j in range(H):
                kj = jnp.zeros((16,), jnp.float32)
                for i2 in range(H):
                    kj = kj + sa[i2] * msplat(i2, j)
                znew = zs[j][pl.ds(o, 16)] + h6 * kj + hv * bzsplat(j)
                zs[j][pl.ds(o, 16)] = znew
                table[pl.ds(tb + j * rpt + o, 16)] = dv * znew

        posts = [make_post2(1.0, 0.5 * HSTEP),
                 make_post2(2.0, 0.5 * HSTEP),
                 make_post2(2.0, HSTEP),
                 post4]

        def step(_s, _):
            for ev in range(4):
                with jax.named_scope("agg1"):
                    agg()
                with jax.named_scope("post1"):
                    plsc.parallel_loop(0, nch, unroll=4)(post1)
                with jax.named_scope("exch1"):
                    exchange(shA)
                with jax.named_scope("agg2"):
                    agg()
                with jax.named_scope("post2"):
                    plsc.parallel_loop(0, nch, unroll=4)(posts[ev])
                with jax.named_scope("exch2"):
                    exchange(shB)
            return ()
        lax.fori_loop(0, NSTEPS, step, ())

        for j in range(H):
            pltpu.sync_copy(sacc[j], sc_hbm.at[pl.ds(j * n_pad + ob, rpt)])
        for j in range(H, 8):
            pltpu.sync_copy(raw[0], sc_hbm.at[pl.ds(j * n_pad + ob, rpt)])

    scratch = ([pltpu.VMEM((H * n_pad,), jnp.float32)] +
               [pltpu.VMEM((rpt,), jnp.float32)] * (4 * H) +
               [pltpu.VMEM((cap,), jnp.int32),
                pltpu.VMEM((cap,), jnp.int32),
                pltpu.VMEM((rpt,), jnp.float32),
                pltpu.VMEM((48 * 16,), jnp.float32)] +
               [pltpu.VMEM_SHARED((H * n_pad,), jnp.float32)] * 2)

    return pl.kernel(
        body,
        out_type=jax.ShapeDtypeStruct((8 * n_pad,), jnp.float32),
        mesh=_MESH,
        compiler_params=_SC_PARAMS,
        scratch_types=scratch,
    )


def _tc_back(sc_ref, w2_ref, x_ref, b2_ref, out_ref):
    s_w2 = lax.dot_general(sc_ref[...], w2_ref[...],
                           (((0,), (0,)), ((), ())),
                           preferred_element_type=jnp.float32)
    out_ref[...] = x_ref[...] + s_w2 + b2_ref[...]


def kernel(x, edge_index, W1, b1, W2, b2):
    n, d = x.shape
    e = edge_index.shape[1]
    h = W1.shape[1]
    assert h == H

    rpt = _round_up(-(-n // NTILES), 16)
    n_pad = rpt * NTILES
    cap = _round_up(int(e * rpt / n * 1.12) + 256, 16)
    chunk = 8000
    assert e % chunk == 0 and chunk % 16 == 0

    src = edge_index[0]
    dst = edge_index[1]

    deg, src_own, dst_own = _make_bucket_kernel(e, n_pad, rpt, cap, chunk)(
        src, dst)

    x_pad = jnp.pad(x, ((0, n_pad - n), (0, 0)))
    w1p = jnp.pad(W1, ((0, 0), (0, 8 - h)))
    z, zt, dinv = pl.pallas_call(
        functools.partial(_tc_front, n),
        out_shape=[jax.ShapeDtypeStruct((8, n_pad), jnp.float32),
                   jax.ShapeDtypeStruct((8, n_pad), jnp.float32),
                   jax.ShapeDtypeStruct((1, n_pad), jnp.float32)],
    )(x_pad, w1p, deg.reshape(1, n_pad))

    m_fuse = W2 @ W1
    bz = b2 @ W1
    cvec = jnp.concatenate([m_fuse.reshape(25), bz, b1,
                            jnp.zeros((48 - 35,), jnp.float32)])
    consts = jnp.broadcast_to(cvec[:, None], (48, 16)).reshape(48 * 16)

    zt_tiled = (zt.reshape(8, NTILES, rpt)[:H]
                .transpose(1, 0, 2).reshape(H * n_pad))
    s_cols = _make_ode_kernel(n_pad, rpt, cap)(
        z.reshape(8 * n_pad), zt_tiled, dinv.reshape(n_pad),
        src_own, dst_own, consts)

    w2p = jnp.pad(W2, ((0, 8 - h), (0, 0)))
    out = pl.pallas_call(
        _tc_back,
        out_shape=jax.ShapeDtypeStruct((n_pad, d), jnp.float32),
    )(s_cols.reshape(8, n_pad), w2p, x_pad, b2.reshape(1, d))
    return out[:n]


# speedup vs baseline: 1.1020x; 1.1020x over previous
import functools

import jax
import jax.numpy as jnp
from jax import lax
from jax.experimental import pallas as pl
from jax.experimental.pallas import tpu as pltpu
from jax.experimental.pallas import tpu_sc as plsc

NSTEPS = 10
HSTEP = 1.0 / NSTEPS
NTILES = 16
H = 5

_MESH = plsc.VectorSubcoreMesh(core_axis_name="c", subcore_axis_name="s",
                               num_cores=1)
_SC_PARAMS = pltpu.CompilerParams(needs_layout_passes=False)


def _round_up(v, m):
    return (v + m - 1) // m * m


def _make_bucket_kernel(e, n_pad, rpt, cap, chunk):
    nvec = chunk // 16

    def body(src_hbm, dst_hbm, deg_hbm, srco_hbm, dsto_hbm,
             srcc_v, dstc_v, srcb_v, dstb_v, deg_v):
        t = lax.axis_index("s")
        lo = t * rpt
        lo_v = jnp.full((16,), lo, jnp.int32)
        hi_v = jnp.full((16,), lo + rpt, jnp.int32)
        ones = jnp.full((16,), 1.0, jnp.float32)
        zeros = jnp.zeros((16,), jnp.float32)

        def zb(i, _):
            deg_v[pl.ds(i * 16, 16)] = zeros
            return ()
        lax.fori_loop(0, rpt // 16, zb, ())

        padsrc = jnp.full((16,), n_pad - 1, jnp.int32)
        zoff = jnp.zeros((16,), jnp.int32)

        def pb(i, _):
            srcb_v[pl.ds(i * 16, 16)] = padsrc
            dstb_v[pl.ds(i * 16, 16)] = zoff
            return ()
        lax.fori_loop(0, cap // 16, pb, ())

        nchunks = e // chunk

        def cb(c, off):
            pltpu.sync_copy(src_hbm.at[pl.ds(c * chunk, chunk)], srcc_v)
            pltpu.sync_copy(dst_hbm.at[pl.ds(c * chunk, chunk)], dstc_v)

            def vb(i, off):
                s = srcc_v[pl.ds(i * 16, 16)]
                d = dstc_v[pl.ds(i * 16, 16)]
                m = (d >= lo_v) & (d < hi_v)
                m = m & (jnp.full((16,), off, jnp.int32) <
                         jnp.full((16,), cap - 16, jnp.int32))
                dloc = d - lo_v
                plsc.addupdate_scatter(deg_v, [dloc], ones, mask=m)
                plsc.store_compressed(srcb_v.at[pl.ds(off, 16)], s, mask=m)
                plsc.store_compressed(dstb_v.at[pl.ds(off, 16)], dloc, mask=m)
                return off + jnp.sum(m.astype(jnp.int32))
            return lax.fori_loop(0, nvec, vb, off)
        lax.fori_loop(0, nchunks, cb, jnp.int32(0))

        pltpu.sync_copy(deg_v, deg_hbm.at[pl.ds(lo, rpt)])
        pltpu.sync_copy(srcb_v, srco_hbm.at[pl.ds(t * cap, cap)])
        pltpu.sync_copy(dstb_v, dsto_hbm.at[pl.ds(t * cap, cap)])

    return pl.kernel(
        body,
        out_type=[jax.ShapeDtypeStruct((n_pad,), jnp.float32),
                  jax.ShapeDtypeStruct((NTILES * cap,), jnp.int32),
                  jax.ShapeDtypeStruct((NTILES * cap,), jnp.int32)],
        mesh=_MESH,
        compiler_params=_SC_PARAMS,
        scratch_types=[pltpu.VMEM((chunk,), jnp.int32),
                       pltpu.VMEM((chunk,), jnp.int32),
                       pltpu.VMEM((cap,), jnp.int32),
                       pltpu.VMEM((cap,), jnp.int32),
                       pltpu.VMEM((rpt,), jnp.float32)],
    )


def _tc_front(n, x_ref, w1_ref, deg_ref, z_ref, zt_ref, dinv_ref):
    n_pad = deg_ref.shape[1]
    z = lax.dot_general(w1_ref[...], x_ref[...],
                        (((0,), (1,)), ((), ())),
                        preferred_element_type=jnp.float32)
    deg = deg_ref[...] + 1.0
    col = lax.broadcasted_iota(jnp.int32, (1, n_pad), 1)
    dinv = jnp.where(col < n, lax.rsqrt(jnp.maximum(deg, 1e-12)), 0.0)
    z_ref[...] = z
    zt_ref[...] = z * dinv
    dinv_ref[...] = dinv


def _make_ode_kernel(n_pad, rpt, cap):
    nev = cap // 16
    nch = rpt // 16

    blk = H * rpt

    def body(z_hbm, zt_hbm, dinv_hbm, srco_hbm, dsto_hbm, consts_hbm,
             sc_hbm, *scr):
        (table,
         raw0, raw1, raw2, raw3, raw4,
         zs0, zs1, zs2, zs3, zs4,
         sa0, sa1, sa2, sa3, sa4,
         ac0, ac1, ac2, ac3, ac4,
         srcb, dstb, dinv_v, cons_v,
         shA, shB) = scr
        raw = (raw0, raw1, raw2, raw3, raw4)
        zs = (zs0, zs1, zs2, zs3, zs4)
        sacc = (sa0, sa1, sa2, sa3, sa4)
        stac = (ac0, ac1, ac2, ac3, ac4)

        t = lax.axis_index("s")
        ob = t * rpt
        tb = t * blk
        zeros = jnp.zeros((16,), jnp.float32)

        pltpu.sync_copy(srco_hbm.at[pl.ds(t * cap, cap)], srcb)
        pltpu.sync_copy(dsto_hbm.at[pl.ds(t * cap, cap)], dstb)
        pltpu.sync_copy(dinv_hbm.at[pl.ds(ob, rpt)], dinv_v)
        pltpu.sync_copy(consts_hbm, cons_v)
        pltpu.sync_copy(zt_hbm, table)
        for j in range(H):
            pltpu.sync_copy(z_hbm.at[pl.ds(j * n_pad + ob, rpt)], zs[j])

        q = rpt // 16
        magic = (1 << 18) // q + 1
        @plsc.parallel_loop(0, cap // 16, unroll=4)
        def _(i):
            s = srcb[pl.ds(i * 16, 16)]
            tile = ((s >> 4) * jnp.full((16,), magic, jnp.int32)) >> 18
            srcb[pl.ds(i * 16, 16)] = s + jnp.full(
                (16,), (H - 1) * rpt, jnp.int32) * tile

        def zinit(i, _):
            o = i * 16
            for j in range(H):
                raw[j][pl.ds(o, 16)] = zeros
                sacc[j][pl.ds(o, 16)] = zeros
                stac[j][pl.ds(o, 16)] = zeros
            return ()
        lax.fori_loop(0, nch, zinit, ())

        def msplat(i, j):
            return cons_v[pl.ds((i * H + j) * 16, 16)]

        def bzsplat(j):
            return cons_v[pl.ds((25 + j) * 16, 16)]

        def b1splat(j):
            return cons_v[pl.ds((30 + j) * 16, 16)]

        def agg():
            @plsc.parallel_loop(0, nev, unroll=8)
            def _(i):
                o = i * 16
                g = srcb[pl.ds(o, 16)]
                d = dstb[pl.ds(o, 16)]
                for j in range(H):
                    gj = g if j == 0 else g + jnp.full((16,), j * rpt,
                                                       jnp.int32)
                    v = plsc.load_gather(table, [gj])
                    plsc.addupdate_scatter(raw[j], [d], v)

        def exchange(shared):
            pltpu.sync_copy(table.at[pl.ds(tb, blk)],
                            shared.at[pl.ds(tb, blk)])
            plsc.subcore_barrier()
            pltpu.sync_copy(shared, table)

        def post1(i):
            o = i * 16
            dv = dinv_v[pl.ds(o, 16)]
            for j in range(H):
                r = raw[j][pl.ds(o, 16)]
                utj = table[pl.ds(tb + j * rpt + o, 16)]
                out1 = dv * (r + utj) + b1splat(j)
                hre = jnp.maximum(out1, 0.0)
                table[pl.ds(tb + j * rpt + o, 16)] = dv * hre
                raw[j][pl.ds(o, 16)] = zeros

        def make_post2(w, cnext):
            wv = jnp.float32(w)
            cv = jnp.float32(cnext)

            def post2(i):
                o = i * 16
                dv = dinv_v[pl.ds(o, 16)]
                out2 = []
                for j in range(H):
                    r = raw[j][pl.ds(o, 16)]
                    htj = table[pl.ds(tb + j * rpt + o, 16)]
                    o2 = dv * (r + htj)
                    out2.append(o2)
                    raw[j][pl.ds(o, 16)] = zeros
                    stac[j][pl.ds(o, 16)] = stac[j][pl.ds(o, 16)] + wv * o2
                for j in range(H):
                    kj = bzsplat(j)
                    for i2 in range(H):
                        kj = kj + out2[i2] * msplat(i2, j)
                    zin = zs[j][pl.ds(o, 16)] + cv * kj
                    table[pl.ds(tb + j * rpt + o, 16)] = dv * zin
            return post2

        def post4(i):
            h6 = jnp.float32(HSTEP / 6.0)
            hv = jnp.float32(HSTEP)
            o = i * 16
            dv = dinv_v[pl.ds(o, 16)]
            sa = []
            for j in range(H):
                r = raw[j][pl.ds(o, 16)]
                htj = table[pl.ds(tb + j * rpt + o, 16)]
                o2 = dv * (r + htj)
                raw[j][pl.ds(o, 16)] = zeros
                sav = stac[j][pl.ds(o, 16)] + o2
                sa.append(sav)
                stac[j][pl.ds(o, 16)] = zeros
                sacc[j][pl.ds(o, 16)] = sacc[j][pl.ds(o, 16)] + h6 * sav
            for j in range(H):
                kj = jnp.zeros((16,), jnp.float32)
                for i2 in range(H):
                    kj = kj + sa[i2] * msplat(i2, j)
                znew = zs[j][pl.ds(o, 16)] + h6 * kj + hv * bzsplat(j)
                zs[j][pl.ds(o, 16)] = znew
                table[pl.ds(tb + j * rpt + o, 16)] = dv * znew

        posts = [make_post2(1.0, 0.5 * HSTEP),
                 make_post2(2.0, 0.5 * HSTEP),
                 make_post2(2.0, HSTEP),
                 post4]

        def step(_s, _):
            for ev in range(4):
                with jax.named_scope("agg1"):
                    agg()
                with jax.named_scope("post1"):
                    plsc.parallel_loop(0, nch, unroll=4)(post1)
                with jax.named_scope("exch1"):
                    exchange(shA)
                with jax.named_scope("agg2"):
                    agg()
                with jax.named_scope("post2"):
                    plsc.parallel_loop(0, nch, unroll=4)(posts[ev])
                with jax.named_scope("exch2"):
                    exchange(shB)
            return ()
        lax.fori_loop(0, NSTEPS, step, ())

        for j in range(H):
            pltpu.sync_copy(sacc[j], sc_hbm.at[pl.ds(j * n_pad + ob, rpt)])
        for j in range(H, 8):
            pltpu.sync_copy(raw[0], sc_hbm.at[pl.ds(j * n_pad + ob, rpt)])

    scratch = ([pltpu.VMEM((H * n_pad,), jnp.float32)] +
               [pltpu.VMEM((rpt,), jnp.float32)] * (4 * H) +
               [pltpu.VMEM((cap,), jnp.int32),
                pltpu.VMEM((cap,), jnp.int32),
                pltpu.VMEM((rpt,), jnp.float32),
                pltpu.VMEM((48 * 16,), jnp.float32)] +
               [pltpu.VMEM_SHARED((H * n_pad,), jnp.float32)] * 2)

    return pl.kernel(
        body,
        out_type=jax.ShapeDtypeStruct((8 * n_pad,), jnp.float32),
        mesh=_MESH,
        compiler_params=_SC_PARAMS,
        scratch_types=scratch,
    )


def _tc_back(sc_ref, w2_ref, x_ref, b2_ref, out_ref):
    s_w2 = lax.dot_general(sc_ref[...], w2_ref[...],
                           (((0,), (0,)), ((), ())),
                           preferred_element_type=jnp.float32)
    out_ref[...] = x_ref[...] + s_w2 + b2_ref[...]


def kernel(x, edge_index, W1, b1, W2, b2):
    n, d = x.shape
    e = edge_index.shape[1]
    h = W1.shape[1]
    assert h == H

    rpt = _round_up(-(-n // NTILES), 16)
    n_pad = rpt * NTILES
    cap = _round_up(int(e * rpt / n * 1.12) + 256, 16)
    chunk = 8000
    assert e % chunk == 0 and chunk % 16 == 0

    src = edge_index[0]
    dst = edge_index[1]

    deg, src_own, dst_own = _make_bucket_kernel(e, n_pad, rpt, cap, chunk)(
        src, dst)

    x_pad = jnp.pad(x, ((0, n_pad - n), (0, 0)))
    w1p = jnp.pad(W1, ((0, 0), (0, 8 - h)))
    z, zt, dinv = pl.pallas_call(
        functools.partial(_tc_front, n),
        out_shape=[jax.ShapeDtypeStruct((8, n_pad), jnp.float32),
                   jax.ShapeDtypeStruct((8, n_pad), jnp.float32),
                   jax.ShapeDtypeStruct((1, n_pad), jnp.float32)],
    )(x_pad, w1p, deg.reshape(1, n_pad))

    m_fuse = W2 @ W1
    bz = b2 @ W1
    cvec = jnp.concatenate([m_fuse.reshape(25), bz, b1,
                            jnp.zeros((48 - 35,), jnp.float32)])
    consts = jnp.broadcast_to(cvec[:, None], (48, 16)).reshape(48 * 16)

    zt_tiled = (zt.reshape(8, NTILES, rpt)[:H]
                .transpose(1, 0, 2).reshape(H * n_pad))
    s_cols = _make_ode_kernel(n_pad, rpt, cap)(
        z.reshape(8 * n_pad), zt_tiled, dinv.reshape(n_pad),
        src_own, dst_own, consts)

    w2p = jnp.pad(W2, ((0, 8 - h), (0, 0)))
    out = pl.pallas_call(
        _tc_back,
        out_shape=jax.ShapeDtypeStruct((n_pad, d), jnp.float32),
    )(s_cols.reshape(8, n_pad), w2p, x_pad, b2.reshape(1, d))
    return out[:n]


# speedup vs baseline: 3.5697x; 3.2393x over previous
import functools

import jax
import jax.numpy as jnp
from jax import lax
from jax.experimental import pallas as pl
from jax.experimental.pallas import tpu as pltpu
from jax.experimental.pallas import tpu_sc as plsc

NSTEPS = 10
HSTEP = 1.0 / NSTEPS
NTILES = 16
H = 5

_MESH = plsc.VectorSubcoreMesh(core_axis_name="c", subcore_axis_name="s",
                               num_cores=1)
_SC_PARAMS = pltpu.CompilerParams(needs_layout_passes=False)


def _round_up(v, m):
    return (v + m - 1) // m * m


def _make_bucket_kernel(e, n_pad, rpt, cap, chunk):
    nvec = chunk // 16

    def body(src_hbm, dst_hbm, deg_hbm, srco_hbm, dsto_hbm, cnt_hbm,
             srcc_v, dstc_v, srcb_v, dstb_v, deg_v, cntb_v):
        t = lax.axis_index("s")
        lo = t * rpt
        lo_v = jnp.full((16,), lo, jnp.int32)
        hi_v = jnp.full((16,), lo + rpt, jnp.int32)
        ones = jnp.full((16,), 1.0, jnp.float32)
        zeros = jnp.zeros((16,), jnp.float32)

        def zb(i, _):
            deg_v[pl.ds(i * 16, 16)] = zeros
            return ()
        lax.fori_loop(0, rpt // 16, zb, ())

        padsrc = jnp.full((16,), n_pad - 1, jnp.int32)
        zoff = jnp.zeros((16,), jnp.int32)

        def pb(i, _):
            srcb_v[pl.ds(i * 16, 16)] = padsrc
            dstb_v[pl.ds(i * 16, 16)] = zoff
            return ()
        lax.fori_loop(0, cap // 16, pb, ())

        nchunks = e // chunk

        def cb(c, off):
            pltpu.sync_copy(src_hbm.at[pl.ds(c * chunk, chunk)], srcc_v)
            pltpu.sync_copy(dst_hbm.at[pl.ds(c * chunk, chunk)], dstc_v)

            def vb(i, off):
                s = srcc_v[pl.ds(i * 16, 16)]
                d = dstc_v[pl.ds(i * 16, 16)]
                m = (d >= lo_v) & (d < hi_v)
                m = m & (jnp.full((16,), off, jnp.int32) <
                         jnp.full((16,), cap - 16, jnp.int32))
                dloc = d - lo_v
                plsc.addupdate_scatter(deg_v, [dloc], ones, mask=m)
                plsc.store_compressed(srcb_v.at[pl.ds(off, 16)], s, mask=m)
                plsc.store_compressed(dstb_v.at[pl.ds(off, 16)], dloc, mask=m)
                return off + jnp.sum(m.astype(jnp.int32))
            return lax.fori_loop(0, nvec, vb, off)
        cnt = lax.fori_loop(0, nchunks, cb, jnp.int32(0))
        cntb_v[...] = jnp.full((16,), cnt, jnp.int32)
        pltpu.sync_copy(cntb_v.at[pl.ds(0, 8)], cnt_hbm.at[pl.ds(t * 8, 8)])

        pltpu.sync_copy(deg_v, deg_hbm.at[pl.ds(lo, rpt)])
        pltpu.sync_copy(srcb_v, srco_hbm.at[pl.ds(t * cap, cap)])
        pltpu.sync_copy(dstb_v, dsto_hbm.at[pl.ds(t * cap, cap)])

    return pl.kernel(
        body,
        out_type=[jax.ShapeDtypeStruct((n_pad,), jnp.float32),
                  jax.ShapeDtypeStruct((NTILES * cap,), jnp.int32),
                  jax.ShapeDtypeStruct((NTILES * cap,), jnp.int32),
                  jax.ShapeDtypeStruct((NTILES * 8,), jnp.int32)],
        mesh=_MESH,
        compiler_params=_SC_PARAMS,
        scratch_types=[pltpu.VMEM((chunk,), jnp.int32),
                       pltpu.VMEM((chunk,), jnp.int32),
                       pltpu.VMEM((cap,), jnp.int32),
                       pltpu.VMEM((cap,), jnp.int32),
                       pltpu.VMEM((rpt,), jnp.float32),
                       pltpu.VMEM((16,), jnp.int32)],
    )


def _tc_front(n, x_ref, w1_ref, deg_ref, z_ref, zt_ref, dinv_ref):
    n_pad = deg_ref.shape[1]
    z = lax.dot_general(w1_ref[...], x_ref[...],
                        (((0,), (1,)), ((), ())),
                        preferred_element_type=jnp.float32)
    deg = deg_ref[...] + 1.0
    col = lax.broadcasted_iota(jnp.int32, (1, n_pad), 1)
    dinv = jnp.where(col < n, lax.rsqrt(jnp.maximum(deg, 1e-12)), 0.0)
    z_ref[...] = z
    zt_ref[...] = z * dinv
    dinv_ref[...] = dinv


def _make_ode_kernel(n_pad, rpt, cap):
    nev = cap // 16
    nch = rpt // 16

    blk = H * rpt

    def body(z_hbm, zt_hbm, dinv_hbm, srco_hbm, dsto_hbm, consts_hbm,
             cnt_hbm, sc_hbm, *scr):
        (table,
         raw0, raw1, raw2, raw3, raw4,
         zs0, zs1, zs2, zs3, zs4,
         sa0, sa1, sa2, sa3, sa4,
         ac0, ac1, ac2, ac3, ac4,
         srcb, dstb, dinv_v, cons_v, cnt_v,
         shA, shB) = scr
        raw = (raw0, raw1, raw2, raw3, raw4)
        zs = (zs0, zs1, zs2, zs3, zs4)
        sacc = (sa0, sa1, sa2, sa3, sa4)
        stac = (ac0, ac1, ac2, ac3, ac4)

        t = lax.axis_index("s")
        ob = t * rpt
        tb = t * blk
        zeros = jnp.zeros((16,), jnp.float32)

        pltpu.sync_copy(srco_hbm.at[pl.ds(t * cap, cap)], srcb)
        pltpu.sync_copy(dsto_hbm.at[pl.ds(t * cap, cap)], dstb)
        pltpu.sync_copy(dinv_hbm.at[pl.ds(ob, rpt)], dinv_v)
        pltpu.sync_copy(consts_hbm, cons_v)
        pltpu.sync_copy(cnt_hbm.at[pl.ds(t * 8, 8)], cnt_v.at[pl.ds(0, 8)])
        pltpu.sync_copy(zt_hbm, table)
        for j in range(H):
            pltpu.sync_copy(z_hbm.at[pl.ds(j * n_pad + ob, rpt)], zs[j])

        q = rpt // 16
        magic = (1 << 18) // q + 1
        @plsc.parallel_loop(0, cap // 16, unroll=4)
        def _(i):
            s = srcb[pl.ds(i * 16, 16)]
            tile = ((s >> 4) * jnp.full((16,), magic, jnp.int32)) >> 18
            srcb[pl.ds(i * 16, 16)] = s + jnp.full(
                (16,), (H - 1) * rpt, jnp.int32) * tile

        def zinit(i, _):
            o = i * 16
            for j in range(H):
                raw[j][pl.ds(o, 16)] = zeros
                sacc[j][pl.ds(o, 16)] = zeros
                stac[j][pl.ds(o, 16)] = zeros
            return ()
        lax.fori_loop(0, nch, zinit, ())

        def msplat(i, j):
            return cons_v[pl.ds((i * H + j) * 16, 16)]

        def bzsplat(j):
            return cons_v[pl.ds((25 + j) * 16, 16)]

        def b1splat(j):
            return cons_v[pl.ds((30 + j) * 16, 16)]

        nev_t = (cnt_v[pl.ds(0, 16)][0] + 15) >> 4

        def agg():
            @plsc.parallel_loop(0, nev_t, unroll=8)
            def _(i):
                o = i * 16
                g = srcb[pl.ds(o, 16)]
                d = dstb[pl.ds(o, 16)]
                for j in range(H):
                    gj = g if j == 0 else g + jnp.full((16,), j * rpt,
                                                       jnp.int32)
                    v = plsc.load_gather(table, [gj])
                    plsc.addupdate_scatter(raw[j], [d], v)

        def exchange(shared):
            pltpu.sync_copy(table.at[pl.ds(tb, blk)],
                            shared.at[pl.ds(tb, blk)])
            plsc.subcore_barrier()
            pltpu.sync_copy(shared, table)

        def post1(i):
            o = i * 16
            dv = dinv_v[pl.ds(o, 16)]
            for j in range(H):
                r = raw[j][pl.ds(o, 16)]
                utj = table[pl.ds(tb + j * rpt + o, 16)]
                out1 = dv * (r + utj) + b1splat(j)
                hre = jnp.maximum(out1, 0.0)
                table[pl.ds(tb + j * rpt + o, 16)] = dv * hre
                raw[j][pl.ds(o, 16)] = zeros

        def make_post2(w, cnext):
            wv = jnp.float32(w)
            cv = jnp.float32(cnext)

            def post2(i):
                o = i * 16
                dv = dinv_v[pl.ds(o, 16)]
                out2 = []
                for j in range(H):
                    r = raw[j][pl.ds(o, 16)]
                    htj = table[pl.ds(tb + j * rpt + o, 16)]
                    o2 = dv * (r + htj)
                    out2.append(o2)
                    raw[j][pl.ds(o, 16)] = zeros
                    stac[j][pl.ds(o, 16)] = stac[j][pl.ds(o, 16)] + wv * o2
                for j in range(H):
                    kj = bzsplat(j)
                    for i2 in range(H):
                        kj = kj + out2[i2] * msplat(i2, j)
                    zin = zs[j][pl.ds(o, 16)] + cv * kj
                    table[pl.ds(tb + j * rpt + o, 16)] = dv * zin
            return post2

        def post4(i):
            h6 = jnp.float32(HSTEP / 6.0)
            hv = jnp.float32(HSTEP)
            o = i * 16
            dv = dinv_v[pl.ds(o, 16)]
            sa = []
            for j in range(H):
                r = raw[j][pl.ds(o, 16)]
                htj = table[pl.ds(tb + j * rpt + o, 16)]
                o2 = dv * (r + htj)
                raw[j][pl.ds(o, 16)] = zeros
                sav = stac[j][pl.ds(o, 16)] + o2
                sa.append(sav)
                stac[j][pl.ds(o, 16)] = zeros
                sacc[j][pl.ds(o, 16)] = sacc[j][pl.ds(o, 16)] + h6 * sav
            for j in range(H):
                kj = jnp.zeros((16,), jnp.float32)
                for i2 in range(H):
                    kj = kj + sa[i2] * msplat(i2, j)
                znew = zs[j][pl.ds(o, 16)] + h6 * kj + hv * bzsplat(j)
                zs[j][pl.ds(o, 16)] = znew
                table[pl.ds(tb + j * rpt + o, 16)] = dv * znew

        posts = [make_post2(1.0, 0.5 * HSTEP),
                 make_post2(2.0, 0.5 * HSTEP),
                 make_post2(2.0, HSTEP),
                 post4]

        def step(_s, _):
            for ev in range(4):
                with jax.named_scope("agg1"):
                    agg()
                with jax.named_scope("post1"):
                    plsc.parallel_loop(0, nch, unroll=4)(post1)
                with jax.named_scope("exch1"):
                    exchange(shA)
                with jax.named_scope("agg2"):
                    agg()
                with jax.named_scope("post2"):
                    plsc.parallel_loop(0, nch, unroll=4)(posts[ev])
                with jax.named_scope("exch2"):
                    exchange(shB)
            return ()
        lax.fori_loop(0, NSTEPS, step, ())

        for j in range(H):
            pltpu.sync_copy(sacc[j], sc_hbm.at[pl.ds(j * n_pad + ob, rpt)])
        for j in range(H, 8):
            pltpu.sync_copy(raw[0], sc_hbm.at[pl.ds(j * n_pad + ob, rpt)])

    scratch = ([pltpu.VMEM((H * n_pad,), jnp.float32)] +
               [pltpu.VMEM((rpt,), jnp.float32)] * (4 * H) +
               [pltpu.VMEM((cap,), jnp.int32),
                pltpu.VMEM((cap,), jnp.int32),
                pltpu.VMEM((rpt,), jnp.float32),
                pltpu.VMEM((48 * 16,), jnp.float32),
                pltpu.VMEM((16,), jnp.int32)] +
               [pltpu.VMEM_SHARED((H * n_pad,), jnp.float32)] * 2)

    return pl.kernel(
        body,
        out_type=jax.ShapeDtypeStruct((8 * n_pad,), jnp.float32),
        mesh=_MESH,
        compiler_params=_SC_PARAMS,
        scratch_types=scratch,
    )


def _tc_back(sc_ref, w2_ref, x_ref, b2_ref, out_ref):
    s_w2 = lax.dot_general(sc_ref[...], w2_ref[...],
                           (((0,), (0,)), ((), ())),
                           preferred_element_type=jnp.float32)
    out_ref[...] = x_ref[...] + s_w2 + b2_ref[...]


def kernel(x, edge_index, W1, b1, W2, b2):
    n, d = x.shape
    e = edge_index.shape[1]
    h = W1.shape[1]
    assert h == H

    rpt = _round_up(-(-n // NTILES), 16)
    n_pad = rpt * NTILES
    cap = _round_up(int(e * rpt / n * 1.12) + 256, 16)
    chunk = 8000
    assert e % chunk == 0 and chunk % 16 == 0

    src = edge_index[0]
    dst = edge_index[1]

    deg, src_own, dst_own, cnt = _make_bucket_kernel(
        e, n_pad, rpt, cap, chunk)(src, dst)

    x_pad = jnp.pad(x, ((0, n_pad - n), (0, 0)))
    w1p = jnp.pad(W1, ((0, 0), (0, 8 - h)))
    z, zt, dinv = pl.pallas_call(
        functools.partial(_tc_front, n),
        out_shape=[jax.ShapeDtypeStruct((8, n_pad), jnp.float32),
                   jax.ShapeDtypeStruct((8, n_pad), jnp.float32),
                   jax.ShapeDtypeStruct((1, n_pad), jnp.float32)],
    )(x_pad, w1p, deg.reshape(1, n_pad))

    m_fuse = W2 @ W1
    bz = b2 @ W1
    cvec = jnp.concatenate([m_fuse.reshape(25), bz, b1,
                            jnp.zeros((48 - 35,), jnp.float32)])
    consts = jnp.broadcast_to(cvec[:, None], (48, 16)).reshape(48 * 16)

    zt_tiled = (zt.reshape(8, NTILES, rpt)[:H]
                .transpose(1, 0, 2).reshape(H * n_pad))
    s_cols = _make_ode_kernel(n_pad, rpt, cap)(
        z.reshape(8 * n_pad), zt_tiled, dinv.reshape(n_pad),
        src_own, dst_own, consts, cnt)

    w2p = jnp.pad(W2, ((0, 8 - h), (0, 0)))
    out = pl.pallas_call(
        _tc_back,
        out_shape=jax.ShapeDtypeStruct((n_pad, d), jnp.float32),
    )(s_cols.reshape(8, n_pad), w2p, x_pad, b2.reshape(1, d))
    return out[:n]


# speedup vs baseline: 3.5982x; 1.0080x over previous
import functools

import jax
import jax.numpy as jnp
from jax import lax
from jax.experimental import pallas as pl
from jax.experimental.pallas import tpu as pltpu
from jax.experimental.pallas import tpu_sc as plsc

NSTEPS = 10
HSTEP = 1.0 / NSTEPS
NTILES = 16
H = 5

_MESH = plsc.VectorSubcoreMesh(core_axis_name="c", subcore_axis_name="s",
                               num_cores=1)
_SC_PARAMS = pltpu.CompilerParams(needs_layout_passes=False)


def _round_up(v, m):
    return (v + m - 1) // m * m


def _make_bucket_kernel(e, n_pad, rpt, cap, chunk):
    nvec = chunk // 16

    def body(src_hbm, dst_hbm, deg_hbm, srco_hbm, dsto_hbm, cnt_hbm,
             srcc_v, dstc_v, srcb_v, dstb_v, deg_v, cntb_v):
        t = lax.axis_index("s")
        lo = t * rpt
        lo_v = jnp.full((16,), lo, jnp.int32)
        hi_v = jnp.full((16,), lo + rpt, jnp.int32)
        ones = jnp.full((16,), 1.0, jnp.float32)
        zeros = jnp.zeros((16,), jnp.float32)

        def zb(i, _):
            deg_v[pl.ds(i * 16, 16)] = zeros
            return ()
        lax.fori_loop(0, rpt // 16, zb, ())

        padsrc = jnp.full((16,), n_pad - 1, jnp.int32)
        zoff = jnp.zeros((16,), jnp.int32)

        def pb(i, _):
            srcb_v[pl.ds(i * 16, 16)] = padsrc
            dstb_v[pl.ds(i * 16, 16)] = zoff
            return ()
        lax.fori_loop(0, cap // 16, pb, ())

        nchunks = e // chunk

        def cb(c, off):
            pltpu.sync_copy(src_hbm.at[pl.ds(c * chunk, chunk)], srcc_v)
            pltpu.sync_copy(dst_hbm.at[pl.ds(c * chunk, chunk)], dstc_v)

            @plsc.parallel_loop(0, nvec, unroll=4, carry=off)
            def vb(i, off):
                s = srcc_v[pl.ds(i * 16, 16)]
                d = dstc_v[pl.ds(i * 16, 16)]
                m = (d >= lo_v) & (d < hi_v)
                m = m & (jnp.full((16,), off, jnp.int32) <
                         jnp.full((16,), cap - 16, jnp.int32))
                dloc = d - lo_v
                plsc.addupdate_scatter(deg_v, [dloc], ones, mask=m)
                plsc.store_compressed(srcb_v.at[pl.ds(off, 16)], s, mask=m)
                plsc.store_compressed(dstb_v.at[pl.ds(off, 16)], dloc, mask=m)
                return off + jnp.sum(m.astype(jnp.int32))
            return vb
        cnt = lax.fori_loop(0, nchunks, cb, jnp.int32(0))
        cntb_v[...] = jnp.full((16,), cnt, jnp.int32)
        pltpu.sync_copy(cntb_v.at[pl.ds(0, 8)], cnt_hbm.at[pl.ds(t * 8, 8)])

        pltpu.sync_copy(deg_v, deg_hbm.at[pl.ds(lo, rpt)])
        pltpu.sync_copy(srcb_v, srco_hbm.at[pl.ds(t * cap, cap)])
        pltpu.sync_copy(dstb_v, dsto_hbm.at[pl.ds(t * cap, cap)])

    return pl.kernel(
        body,
        out_type=[jax.ShapeDtypeStruct((n_pad,), jnp.float32),
                  jax.ShapeDtypeStruct((NTILES * cap,), jnp.int32),
                  jax.ShapeDtypeStruct((NTILES * cap,), jnp.int32),
                  jax.ShapeDtypeStruct((NTILES * 8,), jnp.int32)],
        mesh=_MESH,
        compiler_params=_SC_PARAMS,
        scratch_types=[pltpu.VMEM((chunk,), jnp.int32),
                       pltpu.VMEM((chunk,), jnp.int32),
                       pltpu.VMEM((cap,), jnp.int32),
                       pltpu.VMEM((cap,), jnp.int32),
                       pltpu.VMEM((rpt,), jnp.float32),
                       pltpu.VMEM((16,), jnp.int32)],
    )


def _tc_front(n, x_ref, w1_ref, deg_ref, z_ref, zt_ref, dinv_ref):
    n_pad = deg_ref.shape[1]
    z = lax.dot_general(w1_ref[...], x_ref[...],
                        (((0,), (1,)), ((), ())),
                        preferred_element_type=jnp.float32)
    deg = deg_ref[...] + 1.0
    col = lax.broadcasted_iota(jnp.int32, (1, n_pad), 1)
    dinv = jnp.where(col < n, lax.rsqrt(jnp.maximum(deg, 1e-12)), 0.0)
    z_ref[...] = z
    zt_ref[...] = z * dinv
    dinv_ref[...] = dinv


def _make_ode_kernel(n_pad, rpt, cap):
    nev = cap // 16
    nch = rpt // 16

    blk = H * rpt

    def body(z_hbm, zt_hbm, dinv_hbm, srco_hbm, dsto_hbm, consts_hbm,
             cnt_hbm, sc_hbm, *scr):
        (table,
         raw0, raw1, raw2, raw3, raw4,
         zs0, zs1, zs2, zs3, zs4,
         sa0, sa1, sa2, sa3, sa4,
         ac0, ac1, ac2, ac3, ac4,
         srcb, dstb, dinv_v, cons_v, cnt_v,
         shA, shB) = scr
        raw = (raw0, raw1, raw2, raw3, raw4)
        zs = (zs0, zs1, zs2, zs3, zs4)
        sacc = (sa0, sa1, sa2, sa3, sa4)
        stac = (ac0, ac1, ac2, ac3, ac4)

        t = lax.axis_index("s")
        ob = t * rpt
        tb = t * blk
        zeros = jnp.zeros((16,), jnp.float32)

        pltpu.sync_copy(srco_hbm.at[pl.ds(t * cap, cap)], srcb)
        pltpu.sync_copy(dsto_hbm.at[pl.ds(t * cap, cap)], dstb)
        pltpu.sync_copy(dinv_hbm.at[pl.ds(ob, rpt)], dinv_v)
        pltpu.sync_copy(consts_hbm, cons_v)
        pltpu.sync_copy(cnt_hbm.at[pl.ds(t * 8, 8)], cnt_v.at[pl.ds(0, 8)])
        pltpu.sync_copy(zt_hbm, table)
        for j in range(H):
            pltpu.sync_copy(z_hbm.at[pl.ds(j * n_pad + ob, rpt)], zs[j])

        q = rpt // 16
        magic = (1 << 18) // q + 1
        @plsc.parallel_loop(0, cap // 16, unroll=4)
        def _(i):
            s = srcb[pl.ds(i * 16, 16)]
            tile = ((s >> 4) * jnp.full((16,), magic, jnp.int32)) >> 18
            srcb[pl.ds(i * 16, 16)] = s + jnp.full(
                (16,), (H - 1) * rpt, jnp.int32) * tile

        def zinit(i, _):
            o = i * 16
            for j in range(H):
                raw[j][pl.ds(o, 16)] = zeros
                sacc[j][pl.ds(o, 16)] = zeros
                stac[j][pl.ds(o, 16)] = zeros
            return ()
        lax.fori_loop(0, nch, zinit, ())

        def msplat(i, j):
            return cons_v[pl.ds((i * H + j) * 16, 16)]

        def bzsplat(j):
            return cons_v[pl.ds((25 + j) * 16, 16)]

        def b1splat(j):
            return cons_v[pl.ds((30 + j) * 16, 16)]

        nev_t = (cnt_v[pl.ds(0, 16)][0] + 15) >> 4

        def agg():
            @plsc.parallel_loop(0, nev_t, unroll=8)
            def _(i):
                o = i * 16
                g = srcb[pl.ds(o, 16)]
                d = dstb[pl.ds(o, 16)]
                for j in range(H):
                    gj = g if j == 0 else g + jnp.full((16,), j * rpt,
                                                       jnp.int32)
                    v = plsc.load_gather(table, [gj])
                    plsc.addupdate_scatter(raw[j], [d], v)

        def exchange(shared):
            pltpu.sync_copy(table.at[pl.ds(tb, blk)],
                            shared.at[pl.ds(tb, blk)])
            plsc.subcore_barrier()
            pltpu.sync_copy(shared, table)

        def post1(i):
            o = i * 16
            dv = dinv_v[pl.ds(o, 16)]
            for j in range(H):
                r = raw[j][pl.ds(o, 16)]
                utj = table[pl.ds(tb + j * rpt + o, 16)]
                out1 = dv * (r + utj) + b1splat(j)
                hre = jnp.maximum(out1, 0.0)
                table[pl.ds(tb + j * rpt + o, 16)] = dv * hre
                raw[j][pl.ds(o, 16)] = zeros

        def make_post2(w, cnext):
            wv = jnp.float32(w)
            cv = jnp.float32(cnext)

            def post2(i):
                o = i * 16
                dv = dinv_v[pl.ds(o, 16)]
                out2 = []
                for j in range(H):
                    r = raw[j][pl.ds(o, 16)]
                    htj = table[pl.ds(tb + j * rpt + o, 16)]
                    o2 = dv * (r + htj)
                    out2.append(o2)
                    raw[j][pl.ds(o, 16)] = zeros
                    stac[j][pl.ds(o, 16)] = stac[j][pl.ds(o, 16)] + wv * o2
                for j in range(H):
                    kj = bzsplat(j)
                    for i2 in range(H):
                        kj = kj + out2[i2] * msplat(i2, j)
                    zin = zs[j][pl.ds(o, 16)] + cv * kj
                    table[pl.ds(tb + j * rpt + o, 16)] = dv * zin
            return post2

        def post4(i):
            h6 = jnp.float32(HSTEP / 6.0)
            hv = jnp.float32(HSTEP)
            o = i * 16
            dv = dinv_v[pl.ds(o, 16)]
            sa = []
            for j in range(H):
                r = raw[j][pl.ds(o, 16)]
                htj = table[pl.ds(tb + j * rpt + o, 16)]
                o2 = dv * (r + htj)
                raw[j][pl.ds(o, 16)] = zeros
                sav = stac[j][pl.ds(o, 16)] + o2
                sa.append(sav)
                stac[j][pl.ds(o, 16)] = zeros
                sacc[j][pl.ds(o, 16)] = sacc[j][pl.ds(o, 16)] + h6 * sav
            for j in range(H):
                kj = jnp.zeros((16,), jnp.float32)
                for i2 in range(H):
                    kj = kj + sa[i2] * msplat(i2, j)
                znew = zs[j][pl.ds(o, 16)] + h6 * kj + hv * bzsplat(j)
                zs[j][pl.ds(o, 16)] = znew
                table[pl.ds(tb + j * rpt + o, 16)] = dv * znew

        posts = [make_post2(1.0, 0.5 * HSTEP),
                 make_post2(2.0, 0.5 * HSTEP),
                 make_post2(2.0, HSTEP),
                 post4]

        def step(_s, _):
            for ev in range(4):
                with jax.named_scope("agg1"):
                    agg()
                with jax.named_scope("post1"):
                    plsc.parallel_loop(0, nch, unroll=4)(post1)
                with jax.named_scope("exch1"):
                    exchange(shA)
                with jax.named_scope("agg2"):
                    agg()
                with jax.named_scope("post2"):
                    plsc.parallel_loop(0, nch, unroll=4)(posts[ev])
                with jax.named_scope("exch2"):
                    exchange(shB)
            return ()
        lax.fori_loop(0, NSTEPS, step, ())

        for j in range(H):
            pltpu.sync_copy(sacc[j], sc_hbm.at[pl.ds(j * n_pad + ob, rpt)])
        for j in range(H, 8):
            pltpu.sync_copy(raw[0], sc_hbm.at[pl.ds(j * n_pad + ob, rpt)])

    scratch = ([pltpu.VMEM((H * n_pad,), jnp.float32)] +
               [pltpu.VMEM((rpt,), jnp.float32)] * (4 * H) +
               [pltpu.VMEM((cap,), jnp.int32),
                pltpu.VMEM((cap,), jnp.int32),
                pltpu.VMEM((rpt,), jnp.float32),
                pltpu.VMEM((48 * 16,), jnp.float32),
                pltpu.VMEM((16,), jnp.int32)] +
               [pltpu.VMEM_SHARED((H * n_pad,), jnp.float32)] * 2)

    return pl.kernel(
        body,
        out_type=jax.ShapeDtypeStruct((8 * n_pad,), jnp.float32),
        mesh=_MESH,
        compiler_params=_SC_PARAMS,
        scratch_types=scratch,
    )


def _tc_back(sc_ref, w2_ref, x_ref, b2_ref, out_ref):
    s_w2 = lax.dot_general(sc_ref[...], w2_ref[...],
                           (((0,), (0,)), ((), ())),
                           preferred_element_type=jnp.float32)
    out_ref[...] = x_ref[...] + s_w2 + b2_ref[...]


def kernel(x, edge_index, W1, b1, W2, b2):
    n, d = x.shape
    e = edge_index.shape[1]
    h = W1.shape[1]
    assert h == H

    rpt = _round_up(-(-n // NTILES), 16)
    n_pad = rpt * NTILES
    cap = _round_up(int(e * rpt / n * 1.12) + 256, 16)
    chunk = 8000
    assert e % chunk == 0 and chunk % 16 == 0

    src = edge_index[0]
    dst = edge_index[1]

    deg, src_own, dst_own, cnt = _make_bucket_kernel(
        e, n_pad, rpt, cap, chunk)(src, dst)

    x_pad = jnp.pad(x, ((0, n_pad - n), (0, 0)))
    w1p = jnp.pad(W1, ((0, 0), (0, 8 - h)))
    z, zt, dinv = pl.pallas_call(
        functools.partial(_tc_front, n),
        out_shape=[jax.ShapeDtypeStruct((8, n_pad), jnp.float32),
                   jax.ShapeDtypeStruct((8, n_pad), jnp.float32),
                   jax.ShapeDtypeStruct((1, n_pad), jnp.float32)],
    )(x_pad, w1p, deg.reshape(1, n_pad))

    m_fuse = W2 @ W1
    bz = b2 @ W1
    cvec = jnp.concatenate([m_fuse.reshape(25), bz, b1,
                            jnp.zeros((48 - 35,), jnp.float32)])
    consts = jnp.broadcast_to(cvec[:, None], (48, 16)).reshape(48 * 16)

    zt_tiled = (zt.reshape(8, NTILES, rpt)[:H]
                .transpose(1, 0, 2).reshape(H * n_pad))
    s_cols = _make_ode_kernel(n_pad, rpt, cap)(
        z.reshape(8 * n_pad), zt_tiled, dinv.reshape(n_pad),
        src_own, dst_own, consts, cnt)

    w2p = jnp.pad(W2, ((0, 8 - h), (0, 0)))
    out = pl.pallas_call(
        _tc_back,
        out_shape=jax.ShapeDtypeStruct((n_pad, d), jnp.float32),
    )(s_cols.reshape(8, n_pad), w2p, x_pad, b2.reshape(1, d))
    return out[:n]
